# Initial kernel scaffold; baseline (speedup 1.0000x reference)
import functools

import jax
import jax.numpy as jnp
from jax import lax
from jax.experimental import pallas as pl
from jax.experimental.pallas import tpu as pltpu
from jax.experimental.pallas import tpu_sc as plsc

_LANES = 16
_CHUNK = 4000


@functools.lru_cache(maxsize=None)
def _build(n_nodes, n_edges):
    info = plsc.get_sparse_core_info()
    nc, ns = info.num_cores, info.num_subcores
    nw = nc * ns
    epw = n_edges // nw
    assert n_edges % nw == 0 and epw % _CHUNK == 0
    nch = epw // _CHUNK
    nvec = _CHUNK // _LANES
    mesh = plsc.VectorSubcoreMesh(core_axis_name="c", subcore_axis_name="s")

    @functools.partial(
        pl.kernel,
        out_type=(
            jax.ShapeDtypeStruct((n_edges,), jnp.float32),
            jax.ShapeDtypeStruct((n_edges,), jnp.int32),
        ),
        mesh=mesh,
        scratch_types=[
            pltpu.VMEM((n_nodes,), jnp.int32),
            pltpu.VMEM((_CHUNK,), jnp.int32),
            pltpu.VMEM((_CHUNK,), jnp.int32),
            pltpu.VMEM((_CHUNK,), jnp.float32),
            pltpu.VMEM((_CHUNK,), jnp.int32),
        ],
    )
    def kern(scores_hbm, oid_hbm, src_hbm, dst_hbm, sout_hbm, eqout_hbm,
             table_v, idxa_v, idxb_v, outf_v, outi_v):
        wid = lax.axis_index("s") * nc + lax.axis_index("c")
        base = wid * epw

        pltpu.sync_copy(scores_hbm, table_v)

        def chunk1(i, carry):
            cb = base + i * _CHUNK
            pltpu.sync_copy(src_hbm.at[pl.ds(cb, _CHUNK)], idxa_v)

            def vec(j, c):
                idx = idxa_v[pl.ds(j * _LANES, _LANES)]
                vals = plsc.load_gather(table_v, [idx])
                outf_v[pl.ds(j * _LANES, _LANES)] = plsc.bitcast(
                    vals, jnp.float32)
                return c

            lax.fori_loop(0, nvec, vec, 0)
            pltpu.sync_copy(outf_v, sout_hbm.at[pl.ds(cb, _CHUNK)])
            return carry

        lax.fori_loop(0, nch, chunk1, 0)

        pltpu.sync_copy(oid_hbm, table_v)

        def chunk2(i, carry):
            cb = base + i * _CHUNK
            pltpu.sync_copy(src_hbm.at[pl.ds(cb, _CHUNK)], idxa_v)
            pltpu.sync_copy(dst_hbm.at[pl.ds(cb, _CHUNK)], idxb_v)

            def vec(j, c):
                s = idxa_v[pl.ds(j * _LANES, _LANES)]
                t = idxb_v[pl.ds(j * _LANES, _LANES)]
                a = plsc.load_gather(table_v, [s])
                b = plsc.load_gather(table_v, [t])
                outi_v[pl.ds(j * _LANES, _LANES)] = (a == b).astype(jnp.int32)
                return c

            lax.fori_loop(0, nvec, vec, 0)
            pltpu.sync_copy(outi_v, eqout_hbm.at[pl.ds(cb, _CHUNK)])
            return carry

        lax.fori_loop(0, nch, chunk2, 0)

    return kern


def kernel(scores, object_id, edge_index):
    n_nodes = scores.shape[0]
    n_edges = edge_index.shape[1]
    scores_i = lax.bitcast_convert_type(scores.reshape(-1), jnp.int32)
    src = edge_index[0]
    dst = edge_index[1]
    sout, eqout = _build(n_nodes, n_edges)(scores_i, object_id, src, dst)
    return sout, eqout.astype(jnp.bool_)


# speedup vs baseline: 448.2049x; 448.2049x over previous
import functools

import jax
import jax.numpy as jnp
from jax import lax
from jax.experimental import pallas as pl
from jax.experimental.pallas import tpu as pltpu
from jax.experimental.pallas import tpu_sc as plsc

_LANES = 16
_CHUNK = 4000


@functools.lru_cache(maxsize=None)
def _build(n_nodes, n_edges):
    info = plsc.get_sparse_core_info()
    nc, ns = info.num_cores, info.num_subcores
    nw = nc * ns
    epw = n_edges // nw
    assert n_edges % nw == 0 and epw % _CHUNK == 0
    nch = epw // _CHUNK
    nvec = _CHUNK // _LANES
    mesh = plsc.VectorSubcoreMesh(core_axis_name="c", subcore_axis_name="s")

    @functools.partial(
        pl.kernel,
        out_type=(
            jax.ShapeDtypeStruct((n_edges,), jnp.float32),
            jax.ShapeDtypeStruct((n_edges,), jnp.int32),
        ),
        mesh=mesh,
        compiler_params=pltpu.CompilerParams(needs_layout_passes=False),
        scratch_types=[
            pltpu.VMEM((n_nodes,), jnp.int32),
            pltpu.VMEM((_CHUNK,), jnp.int32),
            pltpu.VMEM((_CHUNK,), jnp.int32),
            pltpu.VMEM((_CHUNK,), jnp.float32),
            pltpu.VMEM((_CHUNK,), jnp.int32),
        ],
    )
    def kern(scores_hbm, oid_hbm, src_hbm, dst_hbm, sout_hbm, eqout_hbm,
             table_v, idxa_v, idxb_v, outf_v, outi_v):
        wid = lax.axis_index("s") * nc + lax.axis_index("c")
        base = wid * epw

        pltpu.sync_copy(scores_hbm, table_v)

        def chunk1(i, carry):
            cb = base + i * _CHUNK
            pltpu.sync_copy(src_hbm.at[pl.ds(cb, _CHUNK)], idxa_v)

            def vec(j, c):
                idx = idxa_v[pl.ds(j * _LANES, _LANES)]
                vals = plsc.load_gather(table_v, [idx])
                outf_v[pl.ds(j * _LANES, _LANES)] = plsc.bitcast(
                    vals, jnp.float32)
                return c

            lax.fori_loop(0, nvec, vec, 0)
            pltpu.sync_copy(outf_v, sout_hbm.at[pl.ds(cb, _CHUNK)])
            return carry

        lax.fori_loop(0, nch, chunk1, 0)

        pltpu.sync_copy(oid_hbm, table_v)

        def chunk2(i, carry):
            cb = base + i * _CHUNK
            pltpu.sync_copy(src_hbm.at[pl.ds(cb, _CHUNK)], idxa_v)
            pltpu.sync_copy(dst_hbm.at[pl.ds(cb, _CHUNK)], idxb_v)

            def vec(j, c):
                s = idxa_v[pl.ds(j * _LANES, _LANES)]
                t = idxb_v[pl.ds(j * _LANES, _LANES)]
                a = plsc.load_gather(table_v, [s])
                b = plsc.load_gather(table_v, [t])
                outi_v[pl.ds(j * _LANES, _LANES)] = (a == b).astype(jnp.int32)
                return c

            lax.fori_loop(0, nvec, vec, 0)
            pltpu.sync_copy(outi_v, eqout_hbm.at[pl.ds(cb, _CHUNK)])
            return carry

        lax.fori_loop(0, nch, chunk2, 0)

    return kern


def kernel(scores, object_id, edge_index):
    n_nodes = scores.shape[0]
    n_edges = edge_index.shape[1]
    scores_i = lax.bitcast_convert_type(scores.reshape(-1), jnp.int32)
    src = edge_index[0]
    dst = edge_index[1]
    sout, eqout = _build(n_nodes, n_edges)(scores_i, object_id, src, dst)
    return sout, eqout.astype(jnp.bool_)


# speedup vs baseline: 597.3211x; 1.3327x over previous
import functools

import jax
import jax.numpy as jnp
from jax import lax
from jax.experimental import pallas as pl
from jax.experimental.pallas import tpu as pltpu
from jax.experimental.pallas import tpu_sc as plsc

_LANES = 16
_CHUNK = 4000


@functools.lru_cache(maxsize=None)
def _build(n_nodes, n_edges):
    info = plsc.get_sparse_core_info()
    nc, ns = info.num_cores, info.num_subcores
    nw = nc * ns
    epw = n_edges // nw
    assert n_edges % nw == 0 and epw % _CHUNK == 0
    nch = epw // _CHUNK
    nvec = _CHUNK // _LANES
    mesh = plsc.VectorSubcoreMesh(core_axis_name="c", subcore_axis_name="s")

    @functools.partial(
        pl.kernel,
        out_type=(
            jax.ShapeDtypeStruct((n_edges,), jnp.float32),
            jax.ShapeDtypeStruct((n_edges,), jnp.int32),
        ),
        mesh=mesh,
        compiler_params=pltpu.CompilerParams(needs_layout_passes=False),
        scratch_types=[
            pltpu.VMEM((n_nodes,), jnp.int32),
            pltpu.VMEM((_CHUNK,), jnp.int32),
            pltpu.VMEM((_CHUNK,), jnp.int32),
            pltpu.VMEM((_CHUNK,), jnp.float32),
            pltpu.VMEM((_CHUNK,), jnp.int32),
        ],
    )
    def kern(scores_hbm, oid_hbm, src_hbm, dst_hbm, sout_hbm, eqout_hbm,
             table_v, idxa_v, idxb_v, outf_v, outi_v):
        wid = lax.axis_index("s") * nc + lax.axis_index("c")
        base = wid * epw

        pltpu.sync_copy(scores_hbm, table_v)

        def chunk1(i, carry):
            cb = base + i * _CHUNK
            pltpu.sync_copy(src_hbm.at[pl.ds(cb, _CHUNK)], idxa_v)

            @plsc.parallel_loop(0, nvec, unroll=8)
            def _(j):
                idx = idxa_v[pl.ds(j * _LANES, _LANES)]
                vals = plsc.load_gather(table_v, [idx])
                outf_v[pl.ds(j * _LANES, _LANES)] = plsc.bitcast(
                    vals, jnp.float32)
            pltpu.sync_copy(outf_v, sout_hbm.at[pl.ds(cb, _CHUNK)])
            return carry

        lax.fori_loop(0, nch, chunk1, 0)

        pltpu.sync_copy(oid_hbm, table_v)

        def chunk2(i, carry):
            cb = base + i * _CHUNK
            pltpu.sync_copy(src_hbm.at[pl.ds(cb, _CHUNK)], idxa_v)
            pltpu.sync_copy(dst_hbm.at[pl.ds(cb, _CHUNK)], idxb_v)

            @plsc.parallel_loop(0, nvec, unroll=8)
            def _(j):
                s = idxa_v[pl.ds(j * _LANES, _LANES)]
                t = idxb_v[pl.ds(j * _LANES, _LANES)]
                a = plsc.load_gather(table_v, [s])
                b = plsc.load_gather(table_v, [t])
                outi_v[pl.ds(j * _LANES, _LANES)] = (a == b).astype(jnp.int32)
            pltpu.sync_copy(outi_v, eqout_hbm.at[pl.ds(cb, _CHUNK)])
            return carry

        lax.fori_loop(0, nch, chunk2, 0)

    return kern


def kernel(scores, object_id, edge_index):
    n_nodes = scores.shape[0]
    n_edges = edge_index.shape[1]
    scores_i = lax.bitcast_convert_type(scores.reshape(-1), jnp.int32)
    src = edge_index[0]
    dst = edge_index[1]
    sout, eqout = _build(n_nodes, n_edges)(scores_i, object_id, src, dst)
    return sout, eqout.astype(jnp.bool_)


# speedup vs baseline: 696.4598x; 1.1660x over previous
import functools

import jax
import jax.numpy as jnp
from jax import lax
from jax.experimental import pallas as pl
from jax.experimental.pallas import tpu as pltpu
from jax.experimental.pallas import tpu_sc as plsc

_LANES = 16
_CHUNK = 10000


@functools.lru_cache(maxsize=None)
def _build(n_nodes, n_edges):
    info = plsc.get_sparse_core_info()
    nc, ns = info.num_cores, info.num_subcores
    nw = nc * ns
    epw = n_edges // nw
    assert n_edges % nw == 0 and epw % _CHUNK == 0
    nch = epw // _CHUNK
    nvec = _CHUNK // _LANES
    mesh = plsc.VectorSubcoreMesh(core_axis_name="c", subcore_axis_name="s")

    @functools.partial(
        pl.kernel,
        out_type=(
            jax.ShapeDtypeStruct((n_edges,), jnp.int32),
            jax.ShapeDtypeStruct((n_edges,), jnp.int32),
        ),
        mesh=mesh,
        compiler_params=pltpu.CompilerParams(needs_layout_passes=False),
        scratch_types=[
            pltpu.VMEM((n_nodes,), jnp.int32),
            pltpu.VMEM((_CHUNK,), jnp.int32),
            pltpu.VMEM((_CHUNK,), jnp.int32),
            pltpu.VMEM((_CHUNK,), jnp.int32),
        ],
    )
    def kern(scores_hbm, oid_hbm, src_hbm, dst_hbm, sout_hbm, eqout_hbm,
             table_v, idxa_v, idxb_v, out_v):
        wid = lax.axis_index("s") * nc + lax.axis_index("c")
        base = wid * epw

        pltpu.sync_copy(scores_hbm, table_v)

        def chunk1(i, carry):
            cb = base + i * _CHUNK
            pltpu.sync_copy(src_hbm.at[pl.ds(cb, _CHUNK)], idxa_v)

            @plsc.parallel_loop(0, nvec, unroll=8)
            def _(j):
                idx = idxa_v[pl.ds(j * _LANES, _LANES)]
                out_v[pl.ds(j * _LANES, _LANES)] = plsc.load_gather(
                    table_v, [idx])
            pltpu.sync_copy(out_v, sout_hbm.at[pl.ds(cb, _CHUNK)])
            return carry

        lax.fori_loop(0, nch, chunk1, 0)

        pltpu.sync_copy(oid_hbm, table_v)

        def chunk2(i, carry):
            cb = base + i * _CHUNK
            pltpu.sync_copy(src_hbm.at[pl.ds(cb, _CHUNK)], idxa_v)
            pltpu.sync_copy(dst_hbm.at[pl.ds(cb, _CHUNK)], idxb_v)

            @plsc.parallel_loop(0, nvec, unroll=8)
            def _(j):
                s = idxa_v[pl.ds(j * _LANES, _LANES)]
                t = idxb_v[pl.ds(j * _LANES, _LANES)]
                a = plsc.load_gather(table_v, [s])
                b = plsc.load_gather(table_v, [t])
                out_v[pl.ds(j * _LANES, _LANES)] = (a == b).astype(jnp.int32)
            pltpu.sync_copy(out_v, eqout_hbm.at[pl.ds(cb, _CHUNK)])
            return carry

        lax.fori_loop(0, nch, chunk2, 0)

    return kern


def kernel(scores, object_id, edge_index):
    n_nodes = scores.shape[0]
    n_edges = edge_index.shape[1]
    scores_i = lax.bitcast_convert_type(scores.reshape(-1), jnp.int32)
    src = edge_index[0]
    dst = edge_index[1]
    sout, eqout = _build(n_nodes, n_edges)(scores_i, object_id, src, dst)
    return (lax.bitcast_convert_type(sout, jnp.float32),
            eqout.astype(jnp.bool_))


# speedup vs baseline: 888.2312x; 1.2754x over previous
import functools

import jax
import jax.numpy as jnp
from jax import lax
from jax.experimental import pallas as pl
from jax.experimental.pallas import tpu as pltpu
from jax.experimental.pallas import tpu_sc as plsc

_LANES = 16
_CHUNK = 4000


@functools.lru_cache(maxsize=None)
def _build(n_nodes, n_edges):
    info = plsc.get_sparse_core_info()
    nc, ns = info.num_cores, info.num_subcores
    nw = nc * ns
    epw = n_edges // nw
    assert n_edges % nw == 0 and epw % (2 * _CHUNK) == 0
    nch = epw // _CHUNK
    nvec = _CHUNK // _LANES
    mesh = plsc.VectorSubcoreMesh(core_axis_name="c", subcore_axis_name="s")

    @functools.partial(
        pl.kernel,
        out_type=(
            jax.ShapeDtypeStruct((n_edges,), jnp.int32),
            jax.ShapeDtypeStruct((n_edges,), jnp.int32),
        ),
        mesh=mesh,
        compiler_params=pltpu.CompilerParams(needs_layout_passes=False),
        scratch_types=[
            pltpu.VMEM((n_nodes,), jnp.int32),
            pltpu.VMEM((_CHUNK,), jnp.int32),
            pltpu.VMEM((_CHUNK,), jnp.int32),
            pltpu.VMEM((_CHUNK,), jnp.int32),
            pltpu.VMEM((_CHUNK,), jnp.int32),
            pltpu.VMEM((_CHUNK,), jnp.int32),
            pltpu.VMEM((_CHUNK,), jnp.int32),
            pltpu.SemaphoreType.DMA,
            pltpu.SemaphoreType.DMA,
            pltpu.SemaphoreType.DMA,
            pltpu.SemaphoreType.DMA,
            pltpu.SemaphoreType.DMA,
            pltpu.SemaphoreType.DMA,
        ],
    )
    def kern(scores_hbm, oid_hbm, src_hbm, dst_hbm, sout_hbm, eqout_hbm,
             table_v, idxa0_v, idxa1_v, idxb0_v, idxb1_v, out0_v, out1_v,
             sa0, sa1, sb0, sb1, so0, so1):
        wid = lax.axis_index("s") * nc + lax.axis_index("c")
        base = wid * epw
        idxa = (idxa0_v, idxa1_v)
        idxb = (idxb0_v, idxb1_v)
        out = (out0_v, out1_v)
        sa = (sa0, sa1)
        sb = (sb0, sb1)
        so = (so0, so1)

        def run_phase(use_dst, out_hbm, compute):
            def fire_in(g, slot):
                cb = base + g * _CHUNK
                pltpu.async_copy(
                    src_hbm.at[pl.ds(cb, _CHUNK)], idxa[slot], sa[slot])
                if use_dst:
                    pltpu.async_copy(
                        dst_hbm.at[pl.ds(cb, _CHUNK)], idxb[slot], sb[slot])

            def wait_in(g, slot):
                cb = base + g * _CHUNK
                pltpu.make_async_copy(
                    src_hbm.at[pl.ds(cb, _CHUNK)], idxa[slot], sa[slot]).wait()
                if use_dst:
                    pltpu.make_async_copy(
                        dst_hbm.at[pl.ds(cb, _CHUNK)], idxb[slot],
                        sb[slot]).wait()

            def wait_out(g, slot):
                cb = base + g * _CHUNK
                pltpu.make_async_copy(
                    out[slot], out_hbm.at[pl.ds(cb, _CHUNK)], so[slot]).wait()

            fire_in(0, 0)

            def body(i, carry):
                for b in range(2):
                    g = 2 * i + b
                    if b == 0:
                        fire_in(g + 1, 1)
                    else:
                        @pl.when(i < nch // 2 - 1)
                        def _():
                            fire_in(g + 1, 0)
                    wait_in(g, b)

                    @pl.when(i >= 1)
                    def _():
                        wait_out(g - 2, b)

                    compute(idxa[b], idxb[b], out[b])
                    cb = base + g * _CHUNK
                    pltpu.async_copy(
                        out[b], out_hbm.at[pl.ds(cb, _CHUNK)], so[b])
                return carry

            lax.fori_loop(0, nch // 2, body, 0)
            wait_out(nch - 2, 0)
            wait_out(nch - 1, 1)

        pltpu.sync_copy(scores_hbm, table_v)

        def compute1(ia, ib, ov):
            @plsc.parallel_loop(0, nvec, unroll=8)
            def _(j):
                idx = ia[pl.ds(j * _LANES, _LANES)]
                ov[pl.ds(j * _LANES, _LANES)] = plsc.load_gather(
                    table_v, [idx])

        run_phase(False, sout_hbm, compute1)

        pltpu.sync_copy(oid_hbm, table_v)

        def compute2(ia, ib, ov):
            @plsc.parallel_loop(0, nvec, unroll=8)
            def _(j):
                s = ia[pl.ds(j * _LANES, _LANES)]
                t = ib[pl.ds(j * _LANES, _LANES)]
                a = plsc.load_gather(table_v, [s])
                b = plsc.load_gather(table_v, [t])
                ov[pl.ds(j * _LANES, _LANES)] = (a == b).astype(jnp.int32)

        run_phase(True, eqout_hbm, compute2)

    return kern


def kernel(scores, object_id, edge_index):
    n_nodes = scores.shape[0]
    n_edges = edge_index.shape[1]
    scores_i = lax.bitcast_convert_type(scores.reshape(-1), jnp.int32)
    src = edge_index[0]
    dst = edge_index[1]
    sout, eqout = _build(n_nodes, n_edges)(scores_i, object_id, src, dst)
    return (lax.bitcast_convert_type(sout, jnp.float32),
            eqout.astype(jnp.bool_))


# speedup vs baseline: 981.3706x; 1.1049x over previous
import functools

import jax
import jax.numpy as jnp
from jax import lax
from jax.experimental import pallas as pl
from jax.experimental.pallas import tpu as pltpu
from jax.experimental.pallas import tpu_sc as plsc

_LANES = 16
_CHUNK1 = 2000
_CHUNK2 = 4000


@functools.lru_cache(maxsize=None)
def _build(n_nodes, n_edges):
    info = plsc.get_sparse_core_info()
    nc, ns = info.num_cores, info.num_subcores
    nw = nc * ns
    epw = n_edges // nw
    assert n_edges % nw == 0
    assert epw % (2 * _CHUNK1) == 0 and epw % (2 * _CHUNK2) == 0
    nch1 = epw // _CHUNK1
    nch2 = epw // _CHUNK2
    nvec1 = _CHUNK1 // _LANES
    nvec2 = _CHUNK2 // _LANES
    mesh = plsc.VectorSubcoreMesh(core_axis_name="c", subcore_axis_name="s")

    @functools.partial(
        pl.kernel,
        out_type=(
            jax.ShapeDtypeStruct((n_edges,), jnp.float32),
            jax.ShapeDtypeStruct((n_edges,), jnp.int32),
        ),
        mesh=mesh,
        compiler_params=pltpu.CompilerParams(needs_layout_passes=False),
        scratch_types=[
            pltpu.VMEM((n_nodes,), jnp.int32),
            pltpu.VMEM((_CHUNK2,), jnp.int32),
            pltpu.VMEM((_CHUNK2,), jnp.int32),
            pltpu.VMEM((_CHUNK2,), jnp.int32),
            pltpu.VMEM((_CHUNK2,), jnp.int32),
            pltpu.VMEM((_CHUNK2,), jnp.int32),
            pltpu.VMEM((_CHUNK2,), jnp.int32),
            pltpu.VMEM((_CHUNK1,), jnp.float32),
            pltpu.VMEM((_CHUNK1,), jnp.float32),
            pltpu.SemaphoreType.DMA,
            pltpu.SemaphoreType.DMA,
            pltpu.SemaphoreType.DMA,
            pltpu.SemaphoreType.DMA,
            pltpu.SemaphoreType.DMA,
            pltpu.SemaphoreType.DMA,
        ],
    )
    def kern(scores_hbm, oid_hbm, ei_hbm, sout_hbm, eqout_hbm,
             table_v, idxa0_v, idxa1_v, idxb0_v, idxb1_v, outi0_v, outi1_v,
             outf0_v, outf1_v, sa0, sa1, sb0, sb1, so0, so1):
        wid = lax.axis_index("s") * nc + lax.axis_index("c")
        base = wid * epw
        idxa = (idxa0_v, idxa1_v)
        idxb = (idxb0_v, idxb1_v)
        outi = (outi0_v, outi1_v)
        outf = (outf0_v, outf1_v)
        sa = (sa0, sa1)
        sb = (sb0, sb1)
        so = (so0, so1)

        def ring(nch, fire_in, wait_in, wait_out, step):
            fire_in(0, 0)

            def body(i, carry):
                for b in range(2):
                    g = 2 * i + b
                    if b == 0:
                        fire_in(g + 1, 1)
                    else:
                        @pl.when(i < nch // 2 - 1)
                        def _():
                            fire_in(g + 1, 0)
                    wait_in(g, b)

                    @pl.when(i >= 1)
                    def _():
                        wait_out(g - 2, b)

                    step(g, b)
                return carry

            lax.fori_loop(0, nch // 2, body, 0)
            wait_out(nch - 2, 0)
            wait_out(nch - 1, 1)

        pltpu.sync_copy(scores_hbm, table_v)

        def fire_in1(g, slot):
            cb = base + g * _CHUNK1
            pltpu.async_copy(ei_hbm.at[pl.ds(cb, _CHUNK1)],
                             idxa[slot].at[pl.ds(0, _CHUNK1)], sa[slot])

        def wait_in1(g, slot):
            cb = base + g * _CHUNK1
            pltpu.make_async_copy(ei_hbm.at[pl.ds(cb, _CHUNK1)],
                                  idxa[slot].at[pl.ds(0, _CHUNK1)],
                                  sa[slot]).wait()

        def wait_out1(g, slot):
            cb = base + g * _CHUNK1
            pltpu.make_async_copy(outf[slot], sout_hbm.at[pl.ds(cb, _CHUNK1)],
                                  so[slot]).wait()

        def step1(g, b):
            @plsc.parallel_loop(0, nvec1, unroll=8)
            def _(j):
                idx = idxa[b][pl.ds(j * _LANES, _LANES)]
                vals = plsc.load_gather(table_v, [idx])
                outf[b][pl.ds(j * _LANES, _LANES)] = plsc.bitcast(
                    vals, jnp.float32)
            cb = base + g * _CHUNK1
            pltpu.async_copy(outf[b], sout_hbm.at[pl.ds(cb, _CHUNK1)], so[b])

        ring(nch1, fire_in1, wait_in1, wait_out1, step1)

        pltpu.sync_copy(oid_hbm, table_v)

        def fire_in2(g, slot):
            cb = base + g * _CHUNK2
            pltpu.async_copy(ei_hbm.at[pl.ds(cb, _CHUNK2)],
                             idxa[slot], sa[slot])
            pltpu.async_copy(ei_hbm.at[pl.ds(n_edges + cb, _CHUNK2)],
                             idxb[slot], sb[slot])

        def wait_in2(g, slot):
            cb = base + g * _CHUNK2
            pltpu.make_async_copy(ei_hbm.at[pl.ds(cb, _CHUNK2)],
                                  idxa[slot], sa[slot]).wait()
            pltpu.make_async_copy(ei_hbm.at[pl.ds(n_edges + cb, _CHUNK2)],
                                  idxb[slot], sb[slot]).wait()

        def wait_out2(g, slot):
            cb = base + g * _CHUNK2
            pltpu.make_async_copy(outi[slot], eqout_hbm.at[pl.ds(cb, _CHUNK2)],
                                  so[slot]).wait()

        def step2(g, b):
            @plsc.parallel_loop(0, nvec2, unroll=8)
            def _(j):
                s = idxa[b][pl.ds(j * _LANES, _LANES)]
                t = idxb[b][pl.ds(j * _LANES, _LANES)]
                a = plsc.load_gather(table_v, [s])
                c = plsc.load_gather(table_v, [t])
                outi[b][pl.ds(j * _LANES, _LANES)] = (a == c).astype(jnp.int32)
            cb = base + g * _CHUNK2
            pltpu.async_copy(outi[b], eqout_hbm.at[pl.ds(cb, _CHUNK2)], so[b])

        ring(nch2, fire_in2, wait_in2, wait_out2, step2)

    return kern


def kernel(scores, object_id, edge_index):
    n_nodes = scores.shape[0]
    n_edges = edge_index.shape[1]
    scores_i = lax.bitcast_convert_type(scores.reshape(-1), jnp.int32)
    ei_flat = edge_index.reshape(-1)
    sout, eqout = _build(n_nodes, n_edges)(scores_i, object_id, ei_flat)
    return sout, eqout.astype(jnp.bool_)
